# baseline (speedup 1.0000x reference)
import functools

import jax
import jax.numpy as jnp
from jax import lax
from jax.experimental import pallas as pl
from jax.experimental.pallas import tpu as pltpu
from jax.experimental.pallas import tpu_sc as plsc

C = 8
V = 1024
H = 2048
B = 4
S = 2048
SCALE = 10.0

NW = 32
TOKENS = S * B
T_PER_W = TOKENS // NW
NBUF = 4
NCHUNK = T_PER_W
IDX_ROWS = T_PER_W * C // 16


def _body(table_hbm, idx_hbm, out_hbm,
          idx_v, gbufs, obufs, gsems, osems):
  nc = 2
  wid = lax.axis_index("s") * nc + lax.axis_index("c")
  row0 = wid * IDX_ROWS
  tok0 = wid * T_PER_W

  pltpu.sync_copy(idx_hbm.at[pl.ds(row0, IDX_ROWS)], idx_v)
  offs = (lax.iota(jnp.int32, 16) & 7) * V

  @pl.loop(0, IDX_ROWS)
  def _(r):
    idx_v[r] = idx_v[r] + offs

  def idx_ref(chunk):
    half = (chunk & 1) * 8
    return idx_v.at[lax.shift_right_logical(chunk, 1), pl.ds(half, 8)]

  def start_gather(chunk, b):
    pltpu.async_copy(table_hbm.at[idx_ref(chunk)], gbufs.at[b], gsems.at[b])

  def wait_gather(b):
    pltpu.make_async_copy(
        table_hbm.at[idx_ref(0)], gbufs.at[b], gsems.at[b]).wait()

  for b in range(NBUF):
    start_gather(b, b)

  @pl.loop(0, NCHUNK, step=NBUF)
  def _(g):
    for b in range(NBUF):
      gc = g + b
      wait_gather(b)
      @pl.when(gc >= NBUF)
      def _():
        pltpu.make_async_copy(
            obufs.at[b], out_hbm.at[pl.ds(tok0, 1)], osems.at[b]).wait()

      gbuf = gbufs.at[b]
      obuf = obufs.at[b]

      @pl.loop(0, H, step=16, unroll=4)
      def _(j):
        col = pl.ds(j, 16)
        v = [gbuf[c, col] for c in range(C)]
        s01 = v[0] + v[1]
        s23 = v[2] + v[3]
        s45 = v[4] + v[5]
        s67 = v[6] + v[7]
        obuf[0, col] = ((s01 + s23) + (s45 + s67)) * SCALE

      pltpu.async_copy(obufs.at[b], out_hbm.at[pl.ds(tok0 + gc, 1)],
                       osems.at[b])

      @pl.when(gc + NBUF < NCHUNK)
      def _():
        start_gather(gc + NBUF, b)

  for b in range(NBUF):
    pltpu.make_async_copy(obufs.at[b], out_hbm.at[pl.ds(tok0, 1)],
                          osems.at[b]).wait()


@jax.jit
def _run(table_flat, idx2d):
  mesh = plsc.VectorSubcoreMesh(core_axis_name="c", subcore_axis_name="s")
  return pl.kernel(
      _body,
      out_type=jax.ShapeDtypeStruct((TOKENS, H), jnp.float32),
      mesh=mesh,
      scratch_types=[
          pltpu.VMEM((IDX_ROWS, 16), jnp.int32),
          pltpu.VMEM((NBUF, C, H), jnp.float32),
          pltpu.VMEM((NBUF, 1, H), jnp.float32),
          pltpu.SemaphoreType.DMA((NBUF,)),
          pltpu.SemaphoreType.DMA((NBUF,)),
      ],
  )(table_flat, idx2d)


def kernel(audio_ids, tables):
  ids = jnp.transpose(audio_ids, (1, 0, 2))
  idx2d = ids.reshape(TOKENS * C // 16, 16)
  table_flat = tables.reshape(C * V, H)
  out = _run(table_flat, idx2d)
  return out.reshape(S, B, H)


# speedup vs baseline: 1.7121x; 1.7121x over previous
import functools

import jax
import jax.numpy as jnp
from jax import lax
from jax.experimental import pallas as pl
from jax.experimental.pallas import tpu as pltpu
from jax.experimental.pallas import tpu_sc as plsc

C = 8
V = 1024
H = 2048
B = 4
S = 2048
SCALE = 10.0

NW = 32
TOKENS = S * B
T_PER_W = TOKENS // NW
NBUF = 4
NCHUNK = T_PER_W
IDX_ROWS = T_PER_W * C // 16


def _body(table_hbm, idx_hbm, out_hbm,
          idx_v, gbufs, obufs, gsems, osems):
  nc = 2
  wid = lax.axis_index("s") * nc + lax.axis_index("c")
  row0 = wid * IDX_ROWS
  tok0 = wid * T_PER_W

  pltpu.sync_copy(idx_hbm.at[pl.ds(row0, IDX_ROWS)], idx_v)
  offs = (lax.iota(jnp.int32, 16) & 7) * V

  @pl.loop(0, IDX_ROWS)
  def _(r):
    idx_v[r] = idx_v[r] + offs

  def idx_ref(chunk):
    half = (chunk & 1) * 8
    return idx_v.at[lax.shift_right_logical(chunk, 1), pl.ds(half, 8)]

  def start_gather(chunk, b):
    pltpu.async_copy(table_hbm.at[idx_ref(chunk)], gbufs.at[b], gsems.at[b])

  def wait_gather(b):
    pltpu.make_async_copy(
        table_hbm.at[idx_ref(0)], gbufs.at[b], gsems.at[b]).wait()

  for b in range(NBUF):
    start_gather(b, b)

  @pl.loop(0, NCHUNK, step=NBUF)
  def _(g):
    for b in range(NBUF):
      gc = g + b
      wait_gather(b)
      @pl.when(gc >= NBUF)
      def _():
        pltpu.make_async_copy(
            obufs.at[b], out_hbm.at[pl.ds(tok0, 1)], osems.at[b]).wait()

      gbuf = gbufs.at[b]
      obuf = obufs.at[b]

      @plsc.parallel_loop(0, H, 16, unroll=4)
      def _(j):
        col = pl.ds(j, 16)
        v = [gbuf[c, col] for c in range(C)]
        s01 = v[0] + v[1]
        s23 = v[2] + v[3]
        s45 = v[4] + v[5]
        s67 = v[6] + v[7]
        obuf[0, col] = ((s01 + s23) + (s45 + s67)) * SCALE

      pltpu.async_copy(obufs.at[b], out_hbm.at[pl.ds(tok0 + gc, 1)],
                       osems.at[b])

      @pl.when(gc + NBUF < NCHUNK)
      def _():
        start_gather(gc + NBUF, b)

  for b in range(NBUF):
    pltpu.make_async_copy(obufs.at[b], out_hbm.at[pl.ds(tok0, 1)],
                          osems.at[b]).wait()


@jax.jit
def _run(table_flat, idx2d):
  mesh = plsc.VectorSubcoreMesh(core_axis_name="c", subcore_axis_name="s")
  return pl.kernel(
      _body,
      out_type=jax.ShapeDtypeStruct((TOKENS, H), jnp.float32),
      mesh=mesh,
      scratch_types=[
          pltpu.VMEM((IDX_ROWS, 16), jnp.int32),
          pltpu.VMEM((NBUF, C, H), jnp.float32),
          pltpu.VMEM((NBUF, 1, H), jnp.float32),
          pltpu.SemaphoreType.DMA((NBUF,)),
          pltpu.SemaphoreType.DMA((NBUF,)),
      ],
  )(table_flat, idx2d)


def kernel(audio_ids, tables):
  ids = jnp.transpose(audio_ids, (1, 0, 2))
  idx2d = ids.reshape(TOKENS * C // 16, 16)
  table_flat = tables.reshape(C * V, H)
  out = _run(table_flat, idx2d)
  return out.reshape(S, B, H)
